# baseline (speedup 1.0000x reference)
import functools

import jax
import jax.numpy as jnp
from jax import lax
from jax.experimental import pallas as pl
from jax.experimental.pallas import tpu as pltpu
from jax.experimental.pallas import tpu_sc as plsc

_NC = 2
_NS = 16
_NW = _NC * _NS
_KS = 80
_KA = 128
_G = 8
_PADROWS = 16


def _acc_rows(n):
    return -(-(n + _PADROWS) // 2048) * 2048


def _sc_gather_s(dst_r, src_r, x):
    nchunk = dst_r.shape[0]
    cp = nchunk // _NW
    n, f = x.shape
    n_acc = _acc_rows(n)
    zrows = n_acc // _NS
    nbuf = 4
    la = 2
    ngroups = -(-cp // _G)
    lb = [min(g * _G, cp - _G) for g in range(ngroups)]

    mesh = plsc.VectorSubcoreMesh(core_axis_name="c", subcore_axis_name="s")

    @functools.partial(
        pl.kernel,
        mesh=mesh,
        compiler_params=pltpu.CompilerParams(use_tc_tiling_on_sc=False),
        out_type=jax.ShapeDtypeStruct((_NC, n_acc, f), jnp.float32),
        scratch_types=(
            [pltpu.VMEM((_G, _KS), jnp.int32) for _ in range(2)]
            + [pltpu.VMEM((_G, _KS), jnp.int32) for _ in range(2)]
            + [pltpu.VMEM((_KS, f), jnp.float32) for _ in range(nbuf)]
            + [pltpu.VMEM_SHARED((n_acc, f), jnp.float32)]
            + [pltpu.SemaphoreType.DMA]
            + [pltpu.SemaphoreType.DMA for _ in range(nbuf)]
            + [pltpu.SemaphoreType.DMA for _ in range(nbuf)]
        ),
    )
    def seg_kernel(dst_h, src_h, x_h, s_out, di0, di1, si0, si1,
                   r0, r1, r2, r3, s_sh, isem, g0, g1, g2, g3,
                   s0, s1, s2, s3):
        cid = lax.axis_index("c")
        sid = lax.axis_index("s")
        wid = cid * _NS + sid
        base = wid * cp

        di = [di0, di1]
        si = [si0, si1]
        rows = [r0, r1, r2, r3]
        gsem = [g0, g1, g2, g3]
        ssem = [s0, s1, s2, s3]

        zvec = jnp.zeros((16,), jnp.float32)
        fv = f // 16

        def fill_rows(i, carry):
            r0[i // fv, pl.ds((i % fv) * 16, 16)] = zvec
            return carry

        lax.fori_loop(0, _KS * fv, fill_rows, 0)

        zbase = sid * zrows
        nfull, rem = divmod(zrows, _KS)
        for j in range(nfull):
            pltpu.sync_copy(r0, s_sh.at[pl.ds(zbase + j * _KS, _KS)])
        if rem:
            pltpu.sync_copy(r0.at[pl.ds(0, rem)],
                            s_sh.at[pl.ds(zbase + nfull * _KS, rem)])
        plsc.subcore_barrier()

        def load_group(g):
            ib = g % 2
            sl = pl.ds(base + lb[g], _G)
            return [pltpu.async_copy(dst_h.at[sl], di[ib], isem),
                    pltpu.async_copy(src_h.at[sl], si[ib], isem)]

        idesc = [None, None]
        idesc[0] = load_group(0)
        for d in idesc[0]:
            d.wait()
        gdesc = [None] * nbuf
        sdesc = [None] * nbuf
        for c in range(cp + la):
            if c < cp:
                b = c % nbuf
                g = c // _G
                if c % _G == 0 and c > 0:
                    for d in idesc[g % 2]:
                        d.wait()
                if sdesc[b] is not None:
                    sdesc[b].wait()
                    sdesc[b] = None
                if c % _G == nbuf - 1 and (g + 1) * _G < cp:
                    idesc[(g + 1) % 2] = load_group(g + 1)
                r = c - lb[g]
                ib = g % 2
                gdesc[b] = (
                    pltpu.async_copy(x_h.at[si[ib].at[r]], rows[b], gsem[b]),
                    ib, r)
            if c >= la:
                pb = (c - la) % nbuf
                gd, ib, r = gdesc[pb]
                gd.wait()
                sdesc[pb] = pltpu.async_copy(
                    rows[pb], s_sh.at[di[ib].at[r]], ssem[pb], add=True)
        for bb in range(nbuf):
            if sdesc[bb] is not None:
                sdesc[bb].wait()
        plsc.subcore_barrier()

        pltpu.sync_copy(s_sh.at[pl.ds(zbase, zrows)],
                        s_out.at[cid, pl.ds(zbase, zrows)])

    return seg_kernel(dst_r, src_r, x)


def _sc_scatter_a(dst_r, ea_r, n):
    nchunk = dst_r.shape[0]
    bond = ea_r.shape[2]
    cp = nchunk // _NW
    extra = nchunk - cp * _NW
    n_acc = _acc_rows(n)
    zrows = n_acc // _NS
    ngroups = -(-cp // _G)
    lb = [min(g * _G, cp - _G) for g in range(ngroups)]

    mesh = plsc.VectorSubcoreMesh(core_axis_name="c", subcore_axis_name="s")

    @functools.partial(
        pl.kernel,
        mesh=mesh,
        compiler_params=pltpu.CompilerParams(use_tc_tiling_on_sc=False),
        out_type=[
            jax.ShapeDtypeStruct((_NC, n_acc, bond), jnp.float32),
            jax.ShapeDtypeStruct((_NC, n_acc, bond), jnp.float32),
        ],
        scratch_types=[
            pltpu.VMEM((_G, _KA), jnp.int32),
            pltpu.VMEM((_G, _KA), jnp.int32),
            pltpu.VMEM((1, _KA), jnp.int32),
            pltpu.VMEM((_KA, bond), jnp.float32),
            pltpu.VMEM((_KA, bond), jnp.float32),
            pltpu.VMEM((_KA, bond), jnp.float32),
            pltpu.VMEM_SHARED((n_acc, bond), jnp.float32),
            pltpu.VMEM_SHARED((n_acc, bond), jnp.float32),
            pltpu.SemaphoreType.DMA,
            pltpu.SemaphoreType.DMA,
            pltpu.SemaphoreType.DMA,
            pltpu.SemaphoreType.DMA,
            pltpu.SemaphoreType.DMA,
        ],
    )
    def seg_kernel(dst_h, ea_h, a_out, d_out,
                   di0, di1, dix, ear0, ear1, ones_v,
                   a_sh, d_sh, isem, g0, g1, s0, s1):
        cid = lax.axis_index("c")
        sid = lax.axis_index("s")
        wid = cid * _NS + sid
        base = wid * cp + jnp.minimum(wid, extra)

        di = [di0, di1]
        ear = [ear0, ear1]
        gsem = [g0, g1]
        ssem = [s0, s1]

        zvec = jnp.zeros((16,), jnp.float32)
        ovec = jnp.ones((16,), jnp.float32)

        def fill_ea(i, carry):
            ear0[i, :] = zvec
            return carry

        lax.fori_loop(0, _KA, fill_ea, 0)

        def fill_ones(i, carry):
            ones_v[i, :] = ovec
            return carry

        lax.fori_loop(0, _KA, fill_ones, 0)

        zbase = sid * zrows
        nfull, rem = divmod(zrows, _KA)
        for j in range(nfull):
            off = zbase + j * _KA
            pltpu.sync_copy(ear0, a_sh.at[pl.ds(off, _KA)])
            pltpu.sync_copy(ear0, d_sh.at[pl.ds(off, _KA)])
        if rem:
            off = zbase + nfull * _KA
            pltpu.sync_copy(ear0.at[pl.ds(0, rem)], a_sh.at[pl.ds(off, rem)])
            pltpu.sync_copy(ear0.at[pl.ds(0, rem)], d_sh.at[pl.ds(off, rem)])
        plsc.subcore_barrier()

        def load_chunk(ch, b):
            return [pltpu.async_copy(ea_h.at[ch], ear[b], gsem[b])]

        def load_group(g):
            ib = g % 2
            return [pltpu.async_copy(dst_h.at[pl.ds(base + lb[g], _G)],
                                     di[ib], isem)]

        idesc = [None, None]
        idesc[0] = load_group(0)
        for d in idesc[0]:
            d.wait()
        gdesc = [None, None]
        sdesc = [None, None]
        for c in range(cp + 1):
            if c < cp:
                b = c % 2
                g = c // _G
                if c % _G == 0 and c > 0:
                    for d in idesc[g % 2]:
                        d.wait()
                if sdesc[b] is not None:
                    for d in sdesc[b]:
                        d.wait()
                    sdesc[b] = None
                if c % _G == 1 and (g + 1) * _G < cp:
                    idesc[(g + 1) % 2] = load_group(g + 1)
                gdesc[b] = (load_chunk(base + c, b), g % 2, c - lb[g])
            if c >= 1:
                pb = (c - 1) % 2
                gds, ib, r = gdesc[pb]
                for d in gds:
                    d.wait()
                dsl = di[ib].at[r]
                sdesc[pb] = [
                    pltpu.async_copy(ear[pb], a_sh.at[dsl], ssem[pb],
                                     add=True),
                    pltpu.async_copy(ones_v, d_sh.at[dsl], ssem[pb],
                                     add=True),
                ]
        for bb in range(2):
            if sdesc[bb] is not None:
                for d in sdesc[bb]:
                    d.wait()

        @pl.when(wid < extra)
        def _tail():
            ch = base + cp
            pltpu.sync_copy(dst_h.at[pl.ds(ch, 1)], dix)
            pltpu.sync_copy(ea_h.at[ch], ear0)
            dsl = dix.at[0]
            pltpu.sync_copy(ear0, a_sh.at[dsl], add=True)
            pltpu.sync_copy(ones_v, d_sh.at[dsl], add=True)

        plsc.subcore_barrier()

        pltpu.sync_copy(a_sh.at[pl.ds(zbase, zrows)],
                        a_out.at[cid, pl.ds(zbase, zrows)])
        pltpu.sync_copy(d_sh.at[pl.ds(zbase, zrows)],
                        d_out.at[cid, pl.ds(zbase, zrows)])

    return seg_kernel(dst_r, ea_r)


def _combine_body(x_ref, s_ref, a_ref, d_ref, ws_ref, wf_ref, wbig_ref,
                  selbig_ref, o_ref):
    xb = x_ref[...]
    bm, c_out = o_ref.shape
    s = s_ref[0] + s_ref[1]
    ap = a_ref[0] + a_ref[1]
    dp = d_ref[0] + d_ref[1]
    u = jnp.dot(ap, wbig_ref[...], preferred_element_type=jnp.float32)
    u = u.reshape(bm, c_out)
    dg = jnp.dot(dp, selbig_ref[...], preferred_element_type=jnp.float32)
    dg = dg.reshape(bm, c_out)
    h = xb * dg + s
    acc = jnp.dot(xb, ws_ref[...], preferred_element_type=jnp.float32)
    acc += jnp.dot(h, wf_ref[...], preferred_element_type=jnp.float32)
    o_ref[...] = acc + u


def _tc_combine(x, s, a_p, d_p, w_s, w_nf, w_nb):
    n, f = x.shape
    bond = w_nb.shape[0]
    c_out = w_s.shape[1]
    bm = 2048
    pk = 128 // bond
    grid = (-(-n // bm),)
    wbig = jax.scipy.linalg.block_diag(*([w_nb] * pk))
    sel = jnp.zeros((bond, c_out), jnp.float32).at[0].set(1.0)
    selbig = jax.scipy.linalg.block_diag(*([sel] * pk))
    return pl.pallas_call(
        _combine_body,
        grid=grid,
        in_specs=[
            pl.BlockSpec((bm, f), lambda i: (i, 0)),
            pl.BlockSpec((_NC, bm, f), lambda i: (0, i, 0)),
            pl.BlockSpec((_NC, bm // 8, 128), lambda i: (0, i, 0)),
            pl.BlockSpec((_NC, bm // 8, 128), lambda i: (0, i, 0)),
            pl.BlockSpec((f, c_out), lambda i: (0, 0)),
            pl.BlockSpec((f, c_out), lambda i: (0, 0)),
            pl.BlockSpec((128, pk * c_out), lambda i: (0, 0)),
            pl.BlockSpec((128, pk * c_out), lambda i: (0, 0)),
        ],
        out_specs=pl.BlockSpec((bm, c_out), lambda i: (i, 0)),
        out_shape=jax.ShapeDtypeStruct((n, c_out), jnp.float32),
    )(x, s, a_p, d_p, w_s, w_nf, wbig, selbig)


def kernel(x, edge_index, edge_attr, w_s, w_n):
    n, f = x.shape
    e = edge_index.shape[1]
    bond = edge_attr.shape[1]

    slab = _KS * _NW
    e_pad = -(-e // slab) * slab
    pad = e_pad - e
    dst = edge_index[0]
    src = edge_index[1]
    if pad:
        fill = n + (jnp.arange(pad, dtype=jnp.int32) % _PADROWS)
        dst = jnp.concatenate([dst, fill])
        src = jnp.concatenate([src, jnp.zeros((pad,), jnp.int32)])
        edge_attr = jnp.concatenate(
            [edge_attr, jnp.zeros((pad, bond), edge_attr.dtype)])

    s = _sc_gather_s(dst.reshape(-1, _KS), src.reshape(-1, _KS), x)
    dst_a, s = lax.optimization_barrier((dst.reshape(-1, _KA), s))
    a, d = _sc_scatter_a(dst_a, edge_attr.reshape(-1, _KA, bond), n)
    a_p = a.reshape(_NC, -1, 128)
    d_p = d.reshape(_NC, -1, 128)
    return _tc_combine(x, s, a_p, d_p, w_s, w_n[:f], w_n[f:])


# speedup vs baseline: 1.0491x; 1.0491x over previous
import functools

import jax
import jax.numpy as jnp
from jax import lax
from jax.experimental import pallas as pl
from jax.experimental.pallas import tpu as pltpu
from jax.experimental.pallas import tpu_sc as plsc

_NC = 2
_NS = 16
_NW = _NC * _NS
_KS = 80
_KA = 128
_G = 8
_PADROWS = 16


def _acc_rows(n):
    return -(-(n + _PADROWS) // 2048) * 2048


def _sc_gather_s(dst_r, src_r, x):
    nchunk = dst_r.shape[0]
    cp = nchunk // _NW
    n, f = x.shape
    n_acc = _acc_rows(n)
    zrows = n_acc // _NS
    nbuf = 4
    la = 2
    ngroups = -(-cp // _G)
    lb = [min(g * _G, cp - _G) for g in range(ngroups)]

    mesh = plsc.VectorSubcoreMesh(core_axis_name="c", subcore_axis_name="s")

    @functools.partial(
        pl.kernel,
        mesh=mesh,
        compiler_params=pltpu.CompilerParams(use_tc_tiling_on_sc=False),
        out_type=jax.ShapeDtypeStruct((_NC, n_acc, f), jnp.float32),
        scratch_types=(
            [pltpu.VMEM((_G, _KS), jnp.int32) for _ in range(2)]
            + [pltpu.VMEM((_G, _KS), jnp.int32) for _ in range(2)]
            + [pltpu.VMEM((_KS, f), jnp.float32) for _ in range(nbuf)]
            + [pltpu.VMEM_SHARED((n_acc, f), jnp.float32)]
            + [pltpu.SemaphoreType.DMA]
            + [pltpu.SemaphoreType.DMA for _ in range(nbuf)]
            + [pltpu.SemaphoreType.DMA for _ in range(nbuf)]
        ),
    )
    def seg_kernel(dst_h, src_h, x_h, s_out, di0, di1, si0, si1,
                   r0, r1, r2, r3, s_sh, isem, g0, g1, g2, g3,
                   s0, s1, s2, s3):
        cid = lax.axis_index("c")
        sid = lax.axis_index("s")
        wid = cid * _NS + sid
        base = wid * cp

        di = [di0, di1]
        si = [si0, si1]
        rows = [r0, r1, r2, r3]
        gsem = [g0, g1, g2, g3]
        ssem = [s0, s1, s2, s3]

        zvec = jnp.zeros((16,), jnp.float32)
        fv = f // 16

        def fill_rows(i, carry):
            r0[i // fv, pl.ds((i % fv) * 16, 16)] = zvec
            return carry

        lax.fori_loop(0, _KS * fv, fill_rows, 0)

        zbase = sid * zrows
        nfull, rem = divmod(zrows, _KS)
        for j in range(nfull):
            pltpu.sync_copy(r0, s_sh.at[pl.ds(zbase + j * _KS, _KS)])
        if rem:
            pltpu.sync_copy(r0.at[pl.ds(0, rem)],
                            s_sh.at[pl.ds(zbase + nfull * _KS, rem)])
        plsc.subcore_barrier()

        def load_group(g):
            ib = g % 2
            sl = pl.ds(base + lb[g], _G)
            return [pltpu.async_copy(dst_h.at[sl], di[ib], isem),
                    pltpu.async_copy(src_h.at[sl], si[ib], isem)]

        idesc = [None, None]
        idesc[0] = load_group(0)
        for d in idesc[0]:
            d.wait()
        gdesc = [None] * nbuf
        sdesc = [None] * nbuf
        for c in range(cp + la):
            if c < cp:
                b = c % nbuf
                g = c // _G
                if c % _G == 0 and c > 0:
                    for d in idesc[g % 2]:
                        d.wait()
                if sdesc[b] is not None:
                    sdesc[b].wait()
                    sdesc[b] = None
                if c % _G == nbuf - 1 and (g + 1) * _G < cp:
                    idesc[(g + 1) % 2] = load_group(g + 1)
                r = c - lb[g]
                ib = g % 2
                gdesc[b] = (
                    pltpu.async_copy(x_h.at[si[ib].at[r]], rows[b], gsem[b]),
                    ib, r)
            if c >= la:
                pb = (c - la) % nbuf
                gd, ib, r = gdesc[pb]
                gd.wait()
                sdesc[pb] = pltpu.async_copy(
                    rows[pb], s_sh.at[di[ib].at[r]], ssem[pb], add=True)
        for bb in range(nbuf):
            if sdesc[bb] is not None:
                sdesc[bb].wait()
        plsc.subcore_barrier()

        pltpu.sync_copy(s_sh.at[pl.ds(zbase, zrows)],
                        s_out.at[cid, pl.ds(zbase, zrows)])

    return seg_kernel(dst_r, src_r, x)


def _sc_scatter_a(dst_r, ea_r, n):
    nchunk = dst_r.shape[0]
    bond = ea_r.shape[2]
    cp = nchunk // _NW
    extra = nchunk - cp * _NW
    n_acc = _acc_rows(n)
    zrows = n_acc // _NS
    ngroups = -(-cp // _G)
    lb = [min(g * _G, cp - _G) for g in range(ngroups)]

    mesh = plsc.VectorSubcoreMesh(core_axis_name="c", subcore_axis_name="s")

    @functools.partial(
        pl.kernel,
        mesh=mesh,
        compiler_params=pltpu.CompilerParams(use_tc_tiling_on_sc=False),
        out_type=[
            jax.ShapeDtypeStruct((_NC, n_acc, bond), jnp.float32),
            jax.ShapeDtypeStruct((_NC, n_acc, bond), jnp.float32),
        ],
        scratch_types=[
            pltpu.VMEM((_G, _KA), jnp.int32),
            pltpu.VMEM((_G, _KA), jnp.int32),
            pltpu.VMEM((1, _KA), jnp.int32),
            pltpu.VMEM((_KA, bond), jnp.float32),
            pltpu.VMEM((_KA, bond), jnp.float32),
            pltpu.VMEM((_KA, bond), jnp.float32),
            pltpu.VMEM((_KA, bond), jnp.float32),
            pltpu.VMEM_SHARED((n_acc, bond), jnp.float32),
            pltpu.VMEM_SHARED((n_acc, bond), jnp.float32),
            pltpu.SemaphoreType.DMA,
            pltpu.SemaphoreType.DMA,
            pltpu.SemaphoreType.DMA,
            pltpu.SemaphoreType.DMA,
            pltpu.SemaphoreType.DMA,
            pltpu.SemaphoreType.DMA,
            pltpu.SemaphoreType.DMA,
        ],
    )
    def seg_kernel(dst_h, ea_h, a_out, d_out,
                   di0, di1, dix, ear0, ear1, ear2, ones_v,
                   a_sh, d_sh, isem, g0, g1, g2, s0, s1, s2):
        cid = lax.axis_index("c")
        sid = lax.axis_index("s")
        wid = cid * _NS + sid
        base = wid * cp + jnp.minimum(wid, extra)

        di = [di0, di1]
        ear = [ear0, ear1, ear2]
        gsem = [g0, g1, g2]
        ssem = [s0, s1, s2]
        nbuf = 3
        la = 2

        zvec = jnp.zeros((16,), jnp.float32)
        ovec = jnp.ones((16,), jnp.float32)

        def fill_ea(i, carry):
            ear0[i, :] = zvec
            return carry

        lax.fori_loop(0, _KA, fill_ea, 0)

        def fill_ones(i, carry):
            ones_v[i, :] = ovec
            return carry

        lax.fori_loop(0, _KA, fill_ones, 0)

        zbase = sid * zrows
        nfull, rem = divmod(zrows, _KA)
        for j in range(nfull):
            off = zbase + j * _KA
            pltpu.sync_copy(ear0, a_sh.at[pl.ds(off, _KA)])
            pltpu.sync_copy(ear0, d_sh.at[pl.ds(off, _KA)])
        if rem:
            off = zbase + nfull * _KA
            pltpu.sync_copy(ear0.at[pl.ds(0, rem)], a_sh.at[pl.ds(off, rem)])
            pltpu.sync_copy(ear0.at[pl.ds(0, rem)], d_sh.at[pl.ds(off, rem)])
        plsc.subcore_barrier()

        def load_chunk(ch, b):
            return [pltpu.async_copy(ea_h.at[ch], ear[b], gsem[b])]

        def load_group(g):
            ib = g % 2
            return [pltpu.async_copy(dst_h.at[pl.ds(base + lb[g], _G)],
                                     di[ib], isem)]

        idesc = [None, None]
        idesc[0] = load_group(0)
        for d in idesc[0]:
            d.wait()
        gdesc = [None] * nbuf
        sdesc = [None] * nbuf
        for c in range(cp + la):
            if c < cp:
                b = c % nbuf
                g = c // _G
                if c % _G == 0 and c > 0:
                    for d in idesc[g % 2]:
                        d.wait()
                if sdesc[b] is not None:
                    for d in sdesc[b]:
                        d.wait()
                    sdesc[b] = None
                if c % _G == nbuf - 1 and (g + 1) * _G < cp:
                    idesc[(g + 1) % 2] = load_group(g + 1)
                gdesc[b] = (load_chunk(base + c, b), g % 2, c - lb[g])
            if c >= la:
                pb = (c - la) % nbuf
                gds, ib, r = gdesc[pb]
                for d in gds:
                    d.wait()
                dsl = di[ib].at[r]
                sdesc[pb] = [
                    pltpu.async_copy(ear[pb], a_sh.at[dsl], ssem[pb],
                                     add=True),
                    pltpu.async_copy(ones_v, d_sh.at[dsl], ssem[pb],
                                     add=True),
                ]
        for bb in range(nbuf):
            if sdesc[bb] is not None:
                for d in sdesc[bb]:
                    d.wait()

        @pl.when(wid < extra)
        def _tail():
            ch = base + cp
            pltpu.sync_copy(dst_h.at[pl.ds(ch, 1)], dix)
            pltpu.sync_copy(ea_h.at[ch], ear0)
            dsl = dix.at[0]
            pltpu.sync_copy(ear0, a_sh.at[dsl], add=True)
            pltpu.sync_copy(ones_v, d_sh.at[dsl], add=True)

        plsc.subcore_barrier()

        pltpu.sync_copy(a_sh.at[pl.ds(zbase, zrows)],
                        a_out.at[cid, pl.ds(zbase, zrows)])
        pltpu.sync_copy(d_sh.at[pl.ds(zbase, zrows)],
                        d_out.at[cid, pl.ds(zbase, zrows)])

    return seg_kernel(dst_r, ea_r)


def _combine_body(x_ref, s_ref, a_ref, d_ref, ws_ref, wf_ref, wbig_ref,
                  selbig_ref, o_ref):
    xb = x_ref[...]
    bm, c_out = o_ref.shape
    s = s_ref[0] + s_ref[1]
    ap = a_ref[0] + a_ref[1]
    dp = d_ref[0] + d_ref[1]
    u = jnp.dot(ap, wbig_ref[...], preferred_element_type=jnp.float32)
    u = u.reshape(bm, c_out)
    dg = jnp.dot(dp, selbig_ref[...], preferred_element_type=jnp.float32)
    dg = dg.reshape(bm, c_out)
    h = xb * dg + s
    acc = jnp.dot(xb, ws_ref[...], preferred_element_type=jnp.float32)
    acc += jnp.dot(h, wf_ref[...], preferred_element_type=jnp.float32)
    o_ref[...] = acc + u


def _tc_combine(x, s, a_p, d_p, w_s, w_nf, w_nb):
    n, f = x.shape
    bond = w_nb.shape[0]
    c_out = w_s.shape[1]
    bm = 2048
    pk = 128 // bond
    grid = (-(-n // bm),)
    wbig = jax.scipy.linalg.block_diag(*([w_nb] * pk))
    sel = jnp.zeros((bond, c_out), jnp.float32).at[0].set(1.0)
    selbig = jax.scipy.linalg.block_diag(*([sel] * pk))
    return pl.pallas_call(
        _combine_body,
        grid=grid,
        in_specs=[
            pl.BlockSpec((bm, f), lambda i: (i, 0)),
            pl.BlockSpec((_NC, bm, f), lambda i: (0, i, 0)),
            pl.BlockSpec((_NC, bm // 8, 128), lambda i: (0, i, 0)),
            pl.BlockSpec((_NC, bm // 8, 128), lambda i: (0, i, 0)),
            pl.BlockSpec((f, c_out), lambda i: (0, 0)),
            pl.BlockSpec((f, c_out), lambda i: (0, 0)),
            pl.BlockSpec((128, pk * c_out), lambda i: (0, 0)),
            pl.BlockSpec((128, pk * c_out), lambda i: (0, 0)),
        ],
        out_specs=pl.BlockSpec((bm, c_out), lambda i: (i, 0)),
        out_shape=jax.ShapeDtypeStruct((n, c_out), jnp.float32),
    )(x, s, a_p, d_p, w_s, w_nf, wbig, selbig)


def kernel(x, edge_index, edge_attr, w_s, w_n):
    n, f = x.shape
    e = edge_index.shape[1]
    bond = edge_attr.shape[1]

    slab = _KS * _NW
    e_pad = -(-e // slab) * slab
    pad = e_pad - e
    dst = edge_index[0]
    src = edge_index[1]
    if pad:
        fill = n + (jnp.arange(pad, dtype=jnp.int32) % _PADROWS)
        dst = jnp.concatenate([dst, fill])
        src = jnp.concatenate([src, jnp.zeros((pad,), jnp.int32)])
        edge_attr = jnp.concatenate(
            [edge_attr, jnp.zeros((pad, bond), edge_attr.dtype)])

    s = _sc_gather_s(dst.reshape(-1, _KS), src.reshape(-1, _KS), x)
    dst_a, s = lax.optimization_barrier((dst.reshape(-1, _KA), s))
    a, d = _sc_scatter_a(dst_a, edge_attr.reshape(-1, _KA, bond), n)
    a_p = a.reshape(_NC, -1, 128)
    d_p = d.reshape(_NC, -1, 128)
    return _tc_combine(x, s, a_p, d_p, w_s, w_n[:f], w_n[f:])
